# baseline (speedup 1.0000x reference)
import functools

import jax
import jax.numpy as jnp
from jax.experimental import pallas as pl
from jax.experimental.pallas import tpu as pltpu

_TEMP = 0.7
_ROWS = 128
_COLS = 100000
_BLOCK = 2048
_NBLK = (_COLS + _BLOCK - 1) // _BLOCK
_TINY = float(jnp.finfo(jnp.float32).tiny)
_NEG_INF = float("-inf")


def _rotl(x, r):
    return (x << jnp.uint32(r)) | (x >> jnp.uint32(32 - r))


def _threefry_bits(ctr):
    ks0 = jnp.uint32(0)
    ks1 = jnp.uint32(42)
    ks2 = ks0 ^ ks1 ^ jnp.uint32(0x1BD11BDA)
    rots = ((13, 15, 26, 6), (17, 29, 16, 24))
    inj = ((ks1, ks2), (ks2, ks0), (ks0, ks1), (ks1, ks2), (ks2, ks0))
    x0 = jnp.full_like(ctr, ks0)
    x1 = ctr + ks1
    for i in range(5):
        for r in rots[i % 2]:
            x0 = x0 + x1
            x1 = _rotl(x1, r)
            x1 = x1 ^ x0
        x0 = x0 + inj[i][0]
        x1 = x1 + inj[i][1] + jnp.uint32(i + 1)
    return x0 ^ x1


def _sample_kernel(x_ref, out_ref, max_ref, arg_ref):
    b = pl.program_id(0)

    col = jax.lax.broadcasted_iota(jnp.int32, (_ROWS, _BLOCK), 1) + b * _BLOCK
    row = jax.lax.broadcasted_iota(jnp.int32, (_ROWS, _BLOCK), 0)
    ctr = (row * _COLS + col).astype(jnp.uint32)

    bits = _threefry_bits(ctr)
    fb = (bits >> jnp.uint32(9)) | jnp.uint32(0x3F800000)
    u = jax.lax.bitcast_convert_type(fb, jnp.float32) - jnp.float32(1.0)
    u = jnp.maximum(jnp.float32(_TINY),
                    u * jnp.float32(1.0 - _TINY) + jnp.float32(_TINY))
    g = -jnp.log(-jnp.log(u))

    s = x_ref[...] * jnp.float32(_TEMP) + g
    s = jnp.where(col < _COLS, s, jnp.float32(_NEG_INF))

    m = jnp.max(s, axis=1, keepdims=True)
    a = jnp.min(jnp.where(s == m, col, jnp.int32(2**31 - 1)),
                axis=1, keepdims=True)

    @pl.when(b == 0)
    def _():
        max_ref[...] = m
        arg_ref[...] = a

    @pl.when(b > 0)
    def _():
        upd = m > max_ref[...]
        arg_ref[...] = jnp.where(upd, a, arg_ref[...])
        max_ref[...] = jnp.maximum(m, max_ref[...])

    @pl.when(b == _NBLK - 1)
    def _():
        out_ref[...] = arg_ref[...]


@jax.jit
def kernel(outputs):
    actions = pl.pallas_call(
        _sample_kernel,
        grid=(_NBLK,),
        in_specs=[pl.BlockSpec((_ROWS, _BLOCK), lambda b: (0, b))],
        out_specs=pl.BlockSpec((_ROWS, 1), lambda b: (0, 0)),
        out_shape=jax.ShapeDtypeStruct((_ROWS, 1), jnp.int32),
        scratch_shapes=[
            pltpu.VMEM((_ROWS, 1), jnp.float32),
            pltpu.VMEM((_ROWS, 1), jnp.int32),
        ],
    )(outputs)
    return actions
